# baseline (speedup 1.0000x reference)
import functools

import jax
import jax.numpy as jnp
from jax.experimental import pallas as pl
from jax.experimental.pallas import tpu as pltpu

_TOPK = 8
_BETA = 1.0
_KB = 512
_QB = 128
_BB = 128
_COL_MASK = _KB - 1
_INT_MAX = jnp.iinfo(jnp.int32).max
_PAD_VAL = 1e17


def _knn_body(x_ref, q_ref, idx_ref, out_ref, top_ref, *, nb, nbq):
    i = pl.program_id(0)
    j = pl.program_id(1)
    rows = pl.ds(j * _QB, _QB)

    @pl.when(i == 0)
    def _init():
        top_ref[rows, :] = jnp.full((_QB, _TOPK), jnp.inf, jnp.float32)

    q = q_ref[...]
    xb = x_ref[...]
    g = jax.lax.dot_general(q, xb, (((1,), (1,)), ((), ())),
                            preferred_element_type=jnp.float32)
    q2 = jnp.sum(q * q, axis=1, keepdims=True)
    x2 = jnp.sum(xb * xb, axis=1)
    d2 = jnp.maximum(q2 + (x2[None, :] - 2.0 * g), 0.0)

    u = jax.lax.bitcast_convert_type(d2, jnp.int32)
    col = jax.lax.broadcasted_iota(jnp.int32, d2.shape, 1)
    key = (u & ~_COL_MASK) | col

    top = top_ref[rows, :]
    neg_inf = jnp.full((_QB, 1), -jnp.inf, jnp.float32)

    def _extract(_, carry):
        key, top = carry
        m = jnp.min(key, axis=1, keepdims=True)
        key = jnp.where(key == m, _INT_MAX, key)
        v = jax.lax.bitcast_convert_type(m & ~_COL_MASK, jnp.float32)
        shifted = jnp.concatenate([neg_inf, top[:, :_TOPK - 1]], axis=1)
        top = jnp.minimum(jnp.maximum(v, shifted), top)
        return key, top

    _, top = jax.lax.fori_loop(0, _TOPK, _extract, (key, top))
    top_ref[rows, :] = top

    @pl.when((i == nb - 1) & (j == nbq - 1))
    def _final():
        w = jnp.exp(-_BETA * jnp.sqrt(top_ref[...] + 1e-12))
        nq = w.shape[0]
        nbb = idx_ref.shape[0] // _BB
        for bi in range(nbb):
            brows = pl.ds(bi * _BB, _BB)
            idx = idx_ref[brows, :]
            q_iota = jax.lax.broadcasted_iota(
                jnp.int32, (_BB, nq), 1)
            onehot = (q_iota == idx).astype(jnp.float32)
            out_ref[brows, :] = jax.lax.dot_general(
                onehot, w, (((1,), (0,)), ((), ())),
                preferred_element_type=jnp.float32)


@jax.jit
def kernel(index_data, query_data, indices):
    k, d = index_data.shape
    q, _ = query_data.shape
    b = indices.shape[0]
    nb = pl.cdiv(k, _KB)
    kp = nb * _KB
    nbq = q // _QB
    if kp != k:
        index_data = jnp.pad(index_data, ((0, kp - k), (0, 0)),
                             constant_values=_PAD_VAL)
    idx2 = indices.reshape(b, 1)

    return pl.pallas_call(
        functools.partial(_knn_body, nb=nb, nbq=nbq),
        grid=(nb, nbq),
        in_specs=[
            pl.BlockSpec((_KB, d), lambda i, j: (i, 0)),
            pl.BlockSpec((_QB, d), lambda i, j: (j, 0)),
            pl.BlockSpec((b, 1), lambda i, j: (0, 0)),
        ],
        out_specs=pl.BlockSpec((b, _TOPK), lambda i, j: (0, 0)),
        out_shape=jax.ShapeDtypeStruct((b, _TOPK), jnp.float32),
        scratch_shapes=[pltpu.VMEM((q, _TOPK), jnp.float32)],
    )(index_data, query_data, idx2)


# speedup vs baseline: 10.0762x; 10.0762x over previous
import functools

import jax
import jax.numpy as jnp
from jax.experimental import pallas as pl
from jax.experimental.pallas import tpu as pltpu

_TOPK = 8
_BETA = 1.0
_KB = 512
_QB = 128
_BB = 128
_COL_MASK = _KB - 1
_INT_MAX = jnp.iinfo(jnp.int32).max
_PAD_VAL = 1e17


def _knn_body(x_ref, q_ref, idx_ref, out_ref, top_ref, *, nb, nbq):
    i = pl.program_id(0)
    j = pl.program_id(1)
    rows = pl.ds(j * _QB, _QB)

    @pl.when(i == 0)
    def _init():
        top_ref[rows, :] = jnp.full((_QB, _TOPK), jnp.inf, jnp.float32)

    q = q_ref[...]
    xt = x_ref[...]
    g = jax.lax.dot_general(q, xt, (((1,), (0,)), ((), ())),
                            preferred_element_type=jnp.float32)
    q2 = jnp.sum(q * q, axis=1, keepdims=True)
    x2 = jnp.sum(xt * xt, axis=0, keepdims=True)
    d2 = jnp.maximum(q2 + (x2 - 2.0 * g), 0.0)

    u = jax.lax.bitcast_convert_type(d2, jnp.int32)
    col = jax.lax.broadcasted_iota(jnp.int32, d2.shape, 1)
    key = (u & ~_COL_MASK) | col

    top = top_ref[rows, :]
    neg_inf = jnp.full((_QB, 1), -jnp.inf, jnp.float32)

    def _extract(_, carry):
        key, top = carry
        m = jnp.min(key, axis=1, keepdims=True)
        key = jnp.where(key == m, _INT_MAX, key)
        v = jax.lax.bitcast_convert_type(m & ~_COL_MASK, jnp.float32)
        shifted = jnp.concatenate([neg_inf, top[:, :_TOPK - 1]], axis=1)
        top = jnp.minimum(jnp.maximum(v, shifted), top)
        return key, top

    _, top = jax.lax.fori_loop(0, _TOPK, _extract, (key, top))
    top_ref[rows, :] = top

    @pl.when((i == nb - 1) & (j == nbq - 1))
    def _final():
        w = jnp.exp(-_BETA * jnp.sqrt(top_ref[...] + 1e-12))
        nq = w.shape[0]
        nbb = idx_ref.shape[0] // _BB
        for bi in range(nbb):
            brows = pl.ds(bi * _BB, _BB)
            idx = idx_ref[brows, :]
            q_iota = jax.lax.broadcasted_iota(
                jnp.int32, (_BB, nq), 1)
            onehot = (q_iota == idx).astype(jnp.float32)
            out_ref[brows, :] = jax.lax.dot_general(
                onehot, w, (((1,), (0,)), ((), ())),
                preferred_element_type=jnp.float32)


@jax.jit
def kernel(index_data, query_data, indices):
    k, d = index_data.shape
    q, _ = query_data.shape
    b = indices.shape[0]
    nb = pl.cdiv(k, _KB)
    kp = nb * _KB
    nbq = q // _QB
    if kp != k:
        index_data = jnp.pad(index_data, ((0, kp - k), (0, 0)),
                             constant_values=_PAD_VAL)
    xt = index_data.T
    idx2 = indices.reshape(b, 1)

    return pl.pallas_call(
        functools.partial(_knn_body, nb=nb, nbq=nbq),
        grid=(nb, nbq),
        in_specs=[
            pl.BlockSpec((d, _KB), lambda i, j: (0, i)),
            pl.BlockSpec((_QB, d), lambda i, j: (j, 0)),
            pl.BlockSpec((b, 1), lambda i, j: (0, 0)),
        ],
        out_specs=pl.BlockSpec((b, _TOPK), lambda i, j: (0, 0)),
        out_shape=jax.ShapeDtypeStruct((b, _TOPK), jnp.float32),
        scratch_shapes=[pltpu.VMEM((q, _TOPK), jnp.float32)],
    )(xt, query_data, idx2)


# speedup vs baseline: 17.3983x; 1.7267x over previous
import functools

import jax
import jax.numpy as jnp
from jax.experimental import pallas as pl
from jax.experimental.pallas import tpu as pltpu

_TOPK = 8
_BETA = 1.0
_KB = 512
_QB = 128
_BB = 128
_COL_MASK = _KB - 1
_INT_MAX = jnp.iinfo(jnp.int32).max
_PAD_VAL = 1e17


def _knn_body(x_ref, q_ref, idx_ref, out_ref, top_ref, *, nb, nbq):
    i = pl.program_id(0)
    j = pl.program_id(1)
    rows = pl.ds(j * _QB, _QB)

    @pl.when(i == 0)
    def _init():
        top_ref[rows, :] = jnp.full((_QB, _TOPK), jnp.inf, jnp.float32)

    q = q_ref[...]
    xt = x_ref[...]
    g = jax.lax.dot_general(q, xt, (((1,), (0,)), ((), ())),
                            preferred_element_type=jnp.float32)
    q2 = jnp.sum(q * q, axis=1, keepdims=True)
    x2 = jnp.sum(xt * xt, axis=0, keepdims=True)
    d2 = jnp.maximum(q2 + (x2 - 2.0 * g), 0.0)

    u = jax.lax.bitcast_convert_type(d2, jnp.int32)
    col = jax.lax.broadcasted_iota(jnp.int32, d2.shape, 1)
    key = (u & ~_COL_MASK) | col

    top = top_ref[rows, :]
    neg_inf = jnp.full((_QB, 1), -jnp.inf, jnp.float32)

    def _val(m):
        return jax.lax.bitcast_convert_type(m & ~_COL_MASK, jnp.float32)

    def _cond(carry):
        _, top, m = carry
        return jnp.any(_val(m) < top[:, _TOPK - 1:])

    def _body(carry):
        key, top, m = carry
        v = _val(m)
        shifted = jnp.concatenate([neg_inf, top[:, :_TOPK - 1]], axis=1)
        top = jnp.minimum(jnp.maximum(v, shifted), top)
        key = jnp.where(key == m, _INT_MAX, key)
        m = jnp.min(key, axis=1, keepdims=True)
        return key, top, m

    m0 = jnp.min(key, axis=1, keepdims=True)
    _, top, _ = jax.lax.while_loop(_cond, _body, (key, top, m0))
    top_ref[rows, :] = top

    @pl.when((i == nb - 1) & (j == nbq - 1))
    def _final():
        w = jnp.exp(-_BETA * jnp.sqrt(top_ref[...] + 1e-12))
        nq = w.shape[0]
        nbb = idx_ref.shape[0] // _BB
        for bi in range(nbb):
            brows = pl.ds(bi * _BB, _BB)
            idx = idx_ref[brows, :]
            q_iota = jax.lax.broadcasted_iota(
                jnp.int32, (_BB, nq), 1)
            onehot = (q_iota == idx).astype(jnp.float32)
            out_ref[brows, :] = jax.lax.dot_general(
                onehot, w, (((1,), (0,)), ((), ())),
                preferred_element_type=jnp.float32)


@jax.jit
def kernel(index_data, query_data, indices):
    k, d = index_data.shape
    q, _ = query_data.shape
    b = indices.shape[0]
    nb = pl.cdiv(k, _KB)
    kp = nb * _KB
    nbq = q // _QB
    if kp != k:
        index_data = jnp.pad(index_data, ((0, kp - k), (0, 0)),
                             constant_values=_PAD_VAL)
    xt = index_data.T
    idx2 = indices.reshape(b, 1)

    return pl.pallas_call(
        functools.partial(_knn_body, nb=nb, nbq=nbq),
        grid=(nb, nbq),
        in_specs=[
            pl.BlockSpec((d, _KB), lambda i, j: (0, i)),
            pl.BlockSpec((_QB, d), lambda i, j: (j, 0)),
            pl.BlockSpec((b, 1), lambda i, j: (0, 0)),
        ],
        out_specs=pl.BlockSpec((b, _TOPK), lambda i, j: (0, 0)),
        out_shape=jax.ShapeDtypeStruct((b, _TOPK), jnp.float32),
        scratch_shapes=[pltpu.VMEM((q, _TOPK), jnp.float32)],
    )(xt, query_data, idx2)


# speedup vs baseline: 33.4360x; 1.9218x over previous
import functools

import jax
import jax.numpy as jnp
from jax.experimental import pallas as pl
from jax.experimental.pallas import tpu as pltpu

_TOPK = 8
_BETA = 1.0
_KB = 2048
_QB = 128
_BB = 128
_COL_MASK = _KB - 1
_INT_MAX = jnp.iinfo(jnp.int32).max
_PAD_VAL = 1e17


def _knn_body(x_ref, q_ref, idx_ref, out_ref, top_ref, *, nb, nbq):
    i = pl.program_id(0)
    j = pl.program_id(1)
    rows = pl.ds(j * _QB, _QB)

    @pl.when(i == 0)
    def _init():
        top_ref[rows, :] = jnp.full((_QB, _TOPK), jnp.inf, jnp.float32)

    q = q_ref[...]
    xt = x_ref[...]
    g = jax.lax.dot_general(q, xt, (((1,), (0,)), ((), ())),
                            preferred_element_type=jnp.float32)
    q2 = jnp.sum(q * q, axis=1, keepdims=True)
    x2 = jnp.sum(xt * xt, axis=0, keepdims=True)
    d2 = jnp.maximum(q2 + (x2 - 2.0 * g), 0.0)

    u = jax.lax.bitcast_convert_type(d2, jnp.int32)
    col = jax.lax.broadcasted_iota(jnp.int32, d2.shape, 1)
    key = (u & ~_COL_MASK) | col

    top = top_ref[rows, :]
    neg_inf = jnp.full((_QB, 1), -jnp.inf, jnp.float32)

    def _val(m):
        return jax.lax.bitcast_convert_type(m & ~_COL_MASK, jnp.float32)

    def _cond(carry):
        _, top, m = carry
        return jnp.any(_val(m) < top[:, _TOPK - 1:])

    def _body(carry):
        key, top, m = carry
        v = _val(m)
        shifted = jnp.concatenate([neg_inf, top[:, :_TOPK - 1]], axis=1)
        top = jnp.minimum(jnp.maximum(v, shifted), top)
        key = jnp.where(key == m, _INT_MAX, key)
        m = jnp.min(key, axis=1, keepdims=True)
        return key, top, m

    m0 = jnp.min(key, axis=1, keepdims=True)
    _, top, _ = jax.lax.while_loop(_cond, _body, (key, top, m0))
    top_ref[rows, :] = top

    @pl.when((i == nb - 1) & (j == nbq - 1))
    def _final():
        w = jnp.exp(-_BETA * jnp.sqrt(top_ref[...] + 1e-12))
        nq = w.shape[0]
        nbb = idx_ref.shape[0] // _BB
        for bi in range(nbb):
            brows = pl.ds(bi * _BB, _BB)
            idx = idx_ref[brows, :]
            q_iota = jax.lax.broadcasted_iota(
                jnp.int32, (_BB, nq), 1)
            onehot = (q_iota == idx).astype(jnp.float32)
            out_ref[brows, :] = jax.lax.dot_general(
                onehot, w, (((1,), (0,)), ((), ())),
                preferred_element_type=jnp.float32)


@jax.jit
def kernel(index_data, query_data, indices):
    k, d = index_data.shape
    q, _ = query_data.shape
    b = indices.shape[0]
    nb = pl.cdiv(k, _KB)
    kp = nb * _KB
    nbq = q // _QB
    if kp != k:
        index_data = jnp.pad(index_data, ((0, kp - k), (0, 0)),
                             constant_values=_PAD_VAL)
    xt = index_data.T
    idx2 = indices.reshape(b, 1)

    return pl.pallas_call(
        functools.partial(_knn_body, nb=nb, nbq=nbq),
        grid=(nb, nbq),
        in_specs=[
            pl.BlockSpec((d, _KB), lambda i, j: (0, i)),
            pl.BlockSpec((_QB, d), lambda i, j: (j, 0)),
            pl.BlockSpec((b, 1), lambda i, j: (0, 0)),
        ],
        out_specs=pl.BlockSpec((b, _TOPK), lambda i, j: (0, 0)),
        out_shape=jax.ShapeDtypeStruct((b, _TOPK), jnp.float32),
        scratch_shapes=[pltpu.VMEM((q, _TOPK), jnp.float32)],
    )(xt, query_data, idx2)


# speedup vs baseline: 34.2945x; 1.0257x over previous
import functools

import jax
import jax.numpy as jnp
from jax.experimental import pallas as pl
from jax.experimental.pallas import tpu as pltpu

_TOPK = 8
_BETA = 1.0
_KB = 4096
_QB = 128
_BB = 128
_COL_MASK = _KB - 1
_INT_MAX = jnp.iinfo(jnp.int32).max
_PAD_VAL = 1e17


def _knn_body(x_ref, q_ref, idx_ref, out_ref, top_ref, *, nb, nbq):
    i = pl.program_id(0)
    j = pl.program_id(1)
    rows = pl.ds(j * _QB, _QB)

    @pl.when(i == 0)
    def _init():
        top_ref[rows, :] = jnp.full((_QB, _TOPK), jnp.inf, jnp.float32)

    q = q_ref[...]
    xt = x_ref[...]
    g = jax.lax.dot_general(q, xt, (((1,), (0,)), ((), ())),
                            preferred_element_type=jnp.float32)
    q2 = jnp.sum(q * q, axis=1, keepdims=True)
    x2 = jnp.sum(xt * xt, axis=0, keepdims=True)
    d2 = jnp.maximum(q2 + (x2 - 2.0 * g), 0.0)

    u = jax.lax.bitcast_convert_type(d2, jnp.int32)
    col = jax.lax.broadcasted_iota(jnp.int32, d2.shape, 1)
    key = (u & ~_COL_MASK) | col

    top = top_ref[rows, :]
    neg_inf = jnp.full((_QB, 1), -jnp.inf, jnp.float32)

    def _val(m):
        return jax.lax.bitcast_convert_type(m & ~_COL_MASK, jnp.float32)

    def _cond(carry):
        _, top, m = carry
        return jnp.any(_val(m) < top[:, _TOPK - 1:])

    def _body(carry):
        key, top, m = carry
        v = _val(m)
        shifted = jnp.concatenate([neg_inf, top[:, :_TOPK - 1]], axis=1)
        top = jnp.minimum(jnp.maximum(v, shifted), top)
        key = jnp.where(key == m, _INT_MAX, key)
        m = jnp.min(key, axis=1, keepdims=True)
        return key, top, m

    m0 = jnp.min(key, axis=1, keepdims=True)
    _, top, _ = jax.lax.while_loop(_cond, _body, (key, top, m0))
    top_ref[rows, :] = top

    @pl.when((i == nb - 1) & (j == nbq - 1))
    def _final():
        w = jnp.exp(-_BETA * jnp.sqrt(top_ref[...] + 1e-12))
        nq = w.shape[0]
        nbb = idx_ref.shape[0] // _BB
        for bi in range(nbb):
            brows = pl.ds(bi * _BB, _BB)
            idx = idx_ref[brows, :]
            q_iota = jax.lax.broadcasted_iota(
                jnp.int32, (_BB, nq), 1)
            onehot = (q_iota == idx).astype(jnp.float32)
            out_ref[brows, :] = jax.lax.dot_general(
                onehot, w, (((1,), (0,)), ((), ())),
                preferred_element_type=jnp.float32)


@jax.jit
def kernel(index_data, query_data, indices):
    k, d = index_data.shape
    q, _ = query_data.shape
    b = indices.shape[0]
    nb = pl.cdiv(k, _KB)
    kp = nb * _KB
    nbq = q // _QB
    if kp != k:
        index_data = jnp.pad(index_data, ((0, kp - k), (0, 0)),
                             constant_values=_PAD_VAL)
    xt = index_data.T
    idx2 = indices.reshape(b, 1)

    return pl.pallas_call(
        functools.partial(_knn_body, nb=nb, nbq=nbq),
        grid=(nb, nbq),
        in_specs=[
            pl.BlockSpec((d, _KB), lambda i, j: (0, i)),
            pl.BlockSpec((_QB, d), lambda i, j: (j, 0)),
            pl.BlockSpec((b, 1), lambda i, j: (0, 0)),
        ],
        out_specs=pl.BlockSpec((b, _TOPK), lambda i, j: (0, 0)),
        out_shape=jax.ShapeDtypeStruct((b, _TOPK), jnp.float32),
        scratch_shapes=[pltpu.VMEM((q, _TOPK), jnp.float32)],
    )(xt, query_data, idx2)


# speedup vs baseline: 38.1473x; 1.1123x over previous
import functools

import jax
import jax.numpy as jnp
from jax.experimental import pallas as pl
from jax.experimental.pallas import tpu as pltpu

_TOPK = 8
_BETA = 1.0
_KB = 4096
_NC = 8
_CW = _KB // _NC
_QB = 128
_BB = 128
_COL_MASK = _CW - 1
_INT_MAX = jnp.iinfo(jnp.int32).max
_PAD_VAL = 1e17


def _knn_body(x_ref, q_ref, idx_ref, out_ref, top_ref, *, nb, nbq):
    i = pl.program_id(0)
    j = pl.program_id(1)
    rows = pl.ds(j * _QB, _QB)

    @pl.when(i == 0)
    def _init():
        top_ref[rows, :] = jnp.full((_QB, _TOPK), jnp.inf, jnp.float32)

    q = q_ref[...]
    xt = x_ref[...]
    g = jax.lax.dot_general(q, xt, (((1,), (0,)), ((), ())),
                            preferred_element_type=jnp.float32)
    q2 = jnp.sum(q * q, axis=1, keepdims=True)
    x2 = jnp.sum(xt * xt, axis=0, keepdims=True)
    d2 = q2 + (x2 - 2.0 * g)

    u = jax.lax.bitcast_convert_type(d2, jnp.int32)
    col = jax.lax.broadcasted_iota(jnp.int32, (_QB, _CW), 1)
    keys = tuple(
        (jax.lax.slice(u, (0, c * _CW), (_QB, (c + 1) * _CW)) & ~_COL_MASK)
        | col
        for c in range(_NC))

    top = top_ref[rows, :]
    neg_inf = jnp.full((_QB, 1), -jnp.inf, jnp.float32)

    def _val(m):
        return jnp.maximum(
            jax.lax.bitcast_convert_type(m & ~_COL_MASK, jnp.float32), 0.0)

    def _mins(keys):
        return tuple(jnp.min(kc, axis=1, keepdims=True) for kc in keys)

    def _cond(carry):
        _, top, rs = carry
        m = functools.reduce(jnp.minimum, rs)
        return jnp.any(_val(m) < top[:, _TOPK - 1:])

    def _body(carry):
        keys, top, rs = carry
        for r in rs:
            v = _val(r)
            shifted = jnp.concatenate([neg_inf, top[:, :_TOPK - 1]], axis=1)
            top = jnp.minimum(jnp.maximum(v, shifted), top)
        keys = tuple(jnp.where(kc == r, _INT_MAX, kc)
                     for kc, r in zip(keys, rs))
        return keys, top, _mins(keys)

    _, top, _ = jax.lax.while_loop(_cond, _body, (keys, top, _mins(keys)))
    top_ref[rows, :] = top

    @pl.when((i == nb - 1) & (j == nbq - 1))
    def _final():
        w = jnp.exp(-_BETA * jnp.sqrt(top_ref[...] + 1e-12))
        nq = w.shape[0]
        nbb = idx_ref.shape[0] // _BB
        for bi in range(nbb):
            brows = pl.ds(bi * _BB, _BB)
            idx = idx_ref[brows, :]
            q_iota = jax.lax.broadcasted_iota(
                jnp.int32, (_BB, nq), 1)
            onehot = (q_iota == idx).astype(jnp.float32)
            out_ref[brows, :] = jax.lax.dot_general(
                onehot, w, (((1,), (0,)), ((), ())),
                preferred_element_type=jnp.float32)


@jax.jit
def kernel(index_data, query_data, indices):
    k, d = index_data.shape
    q, _ = query_data.shape
    b = indices.shape[0]
    nb = pl.cdiv(k, _KB)
    kp = nb * _KB
    nbq = q // _QB
    if kp != k:
        index_data = jnp.pad(index_data, ((0, kp - k), (0, 0)),
                             constant_values=_PAD_VAL)
    xt = index_data.T
    idx2 = indices.reshape(b, 1)

    return pl.pallas_call(
        functools.partial(_knn_body, nb=nb, nbq=nbq),
        grid=(nb, nbq),
        in_specs=[
            pl.BlockSpec((d, _KB), lambda i, j: (0, i)),
            pl.BlockSpec((_QB, d), lambda i, j: (j, 0)),
            pl.BlockSpec((b, 1), lambda i, j: (0, 0)),
        ],
        out_specs=pl.BlockSpec((b, _TOPK), lambda i, j: (0, 0)),
        out_shape=jax.ShapeDtypeStruct((b, _TOPK), jnp.float32),
        scratch_shapes=[pltpu.VMEM((q, _TOPK), jnp.float32)],
    )(xt, query_data, idx2)
